# baseline (speedup 1.0000x reference)
import functools

import jax
import jax.numpy as jnp
from jax import lax
from jax.experimental import pallas as pl
from jax.experimental.pallas import tpu as pltpu
from jax.experimental.pallas import tpu_sc as plsc

B = 16
N = 16384
M = 64
L = 16
NC = 2
NS = 16
NW = NC * NS
APW = (B * N) // NW
CH = 1024
NCHUNK = APW // CH
GRP = CH // L

_mesh = plsc.VectorSubcoreMesh(core_axis_name="c", subcore_axis_name="s")


def _body(face_h, lmp_h, gzp_h, tbox_h, tlm_h, tgz_h, mat_h, lab_h, out_h,
          face_v0, face_v1, lmp_v0, lmp_v1, gzp_v0, gzp_v1,
          mat_v0, mat_v1, lab_v0, lab_v1,
          tbox_v, tlm_v, tgz_v, out_v, sem0, sem1):
    cid = lax.axis_index("c")
    sid = lax.axis_index("s")
    wid = sid * NC + cid
    img = wid // 2
    half = wid % 2
    n0 = half * APW

    pltpu.sync_copy(tbox_h.at[img], tbox_v)
    pltpu.sync_copy(tlm_h.at[img], tlm_v)
    pltpu.sync_copy(tgz_h.at[img], tgz_v)

    bufs = ((face_v0, lmp_v0, gzp_v0, mat_v0, lab_v0, sem0),
            (face_v1, lmp_v1, gzp_v1, mat_v1, lab_v1, sem1))

    def start(c, slot):
        fv, lv, gv, mv, bv, sem = bufs[slot]
        base = n0 + c * CH
        return [
            pltpu.async_copy(face_h.at[img, pl.ds(base, CH), :], fv, sem),
            pltpu.async_copy(lmp_h.at[img, pl.ds(base, CH), :], lv, sem),
            pltpu.async_copy(gzp_h.at[img, pl.ds(base, CH), :], gv, sem),
            pltpu.async_copy(mat_h.at[img, pl.ds(base, CH)], mv, sem),
            pltpu.async_copy(lab_h.at[img, pl.ds(base, CH)], bv, sem),
        ]

    iota = jnp.arange(L, dtype=jnp.int32)
    jvs = [jnp.full((L,), j, jnp.int32) for j in range(10)]

    def compute(slot, accs):
        fv, lv, gv, mv, bv, _ = bufs[slot]

        def group(g, accs):
            abce, abox, alm, agz = accs
            a_idx = g * L + iota
            m = plsc.load_gather(mv, [a_idx])
            lab = plsc.load_gather(bv, [a_idx])
            maskf = jnp.where(lab > 0.0, 1.0, 0.0).astype(jnp.float32)

            def sl1(pred_ref, tbl_ref, ncomp):
                sq = jnp.zeros((L,), jnp.float32)
                mx = jnp.zeros((L,), jnp.float32)
                for j in range(ncomp):
                    p = plsc.load_gather(pred_ref, [a_idx, jvs[j]])
                    t = plsc.load_gather(tbl_ref, [m, jvs[j]])
                    d = jnp.abs(p - t)
                    dm = jnp.minimum(d, 1.0)
                    sq = sq + dm * dm
                    mx = mx + jnp.maximum(d, 1.0)
                return (0.5 * sq + mx - float(ncomp)) * maskf

            abox = abox + sl1(fv, tbox_v, 4)
            alm = alm + sl1(lv, tlm_v, 10)
            agz = agz + sl1(gv, tgz_v, 2)

            x = plsc.load_gather(fv, [a_idx, jvs[4]])
            u = jnp.exp(-jnp.abs(x))
            v = u / (u + 2.0)
            v2 = v * v
            sp = v * (2.0 + v2 * (2.0 / 3.0 + v2 * (2.0 / 5.0
                      + v2 * (2.0 / 7.0 + v2 * (2.0 / 9.0)))))
            abce = abce + (jnp.maximum(x, 0.0) - x * lab + sp)
            return (abce, abox, alm, agz)

        return lax.fori_loop(0, GRP, group, accs, unroll=4)

    z = jnp.zeros((L,), jnp.float32)
    accs = (z, z, z, z)
    pending = start(0, 0)
    for c in range(NCHUNK):
        for hd in pending:
            hd.wait()
        if c + 1 < NCHUNK:
            nxt = start(c + 1, (c + 1) % 2)
        else:
            nxt = []
        accs = compute(c % 2, accs)
        pending = nxt

    out_v[0, :] = accs[0]
    out_v[1, :] = accs[1]
    out_v[2, :] = accs[2]
    out_v[3, :] = accs[3]
    pltpu.sync_copy(out_v, out_h.at[wid])


_sc_loss = functools.partial(
    pl.kernel,
    out_type=jax.ShapeDtypeStruct((NW, 4, L), jnp.float32),
    mesh=_mesh,
    scratch_types=[
        pltpu.VMEM((CH, 5), jnp.float32),
        pltpu.VMEM((CH, 5), jnp.float32),
        pltpu.VMEM((CH, 10), jnp.float32),
        pltpu.VMEM((CH, 10), jnp.float32),
        pltpu.VMEM((CH, 2), jnp.float32),
        pltpu.VMEM((CH, 2), jnp.float32),
        pltpu.VMEM((CH,), jnp.int32),
        pltpu.VMEM((CH,), jnp.int32),
        pltpu.VMEM((CH,), jnp.float32),
        pltpu.VMEM((CH,), jnp.float32),
        pltpu.VMEM((M, 4), jnp.float32),
        pltpu.VMEM((M, 10), jnp.float32),
        pltpu.VMEM((M, 2), jnp.float32),
        pltpu.VMEM((4, L), jnp.float32),
        pltpu.SemaphoreType.DMA,
        pltpu.SemaphoreType.DMA,
    ],
    compiler_params=pltpu.CompilerParams(
        needs_layout_passes=False, use_tc_tiling_on_sc=False),
)(_body)


def kernel(face_preds, landmark_preds, gaze_preds, boxes, landmarks, gaze,
           matches, labels):
    part = _sc_loss(
        face_preds,
        landmark_preds,
        gaze_preds,
        boxes,
        landmarks,
        gaze,
        matches.astype(jnp.int32),
        labels,
    )
    s = jnp.sum(part, axis=(0, 2))
    face_loss = s[0] + s[1]
    landmark_loss = s[2]
    gaze_loss = s[3]
    total_loss = face_loss + landmark_loss + gaze_loss
    return (total_loss, face_loss, landmark_loss, gaze_loss)


# speedup vs baseline: 5.8529x; 5.8529x over previous
import functools

import jax
import jax.numpy as jnp
from jax import lax
from jax.experimental import pallas as pl
from jax.experimental.pallas import tpu as pltpu
from jax.experimental.pallas import tpu_sc as plsc

B = 16
N = 16384
M = 64
L = 16
NC = 2
NS = 16
NW = NC * NS
APW = (B * N) // NW
CH = 2048
NCHUNK = APW // CH
GRP = CH // L

_mesh = plsc.VectorSubcoreMesh(core_axis_name="c", subcore_axis_name="s")


def _body(face_h, lmp_h, gzp_h, tbox_h, tlm_h, tgz_h, mat_h, lab_h, out_h,
          face_v0, face_v1, lmp_v0, lmp_v1, gzp_v0, gzp_v1,
          mat_v0, mat_v1, lab_v0, lab_v1,
          tbox_v, tlm_v, tgz_v, out_v, sem0, sem1):
    cid = lax.axis_index("c")
    sid = lax.axis_index("s")
    wid = sid * NC + cid
    img = wid // 2
    half = wid % 2
    n0 = half * APW

    pltpu.sync_copy(tbox_h.at[img], tbox_v)
    pltpu.sync_copy(tlm_h.at[:, img, :], tlm_v)
    pltpu.sync_copy(tgz_h.at[img], tgz_v)

    bufs = ((face_v0, lmp_v0, gzp_v0, mat_v0, lab_v0, sem0),
            (face_v1, lmp_v1, gzp_v1, mat_v1, lab_v1, sem1))

    def start(c, slot):
        fv, lv, gv, mv, bv, sem = bufs[slot]
        base = n0 + c * CH
        hs = []
        for j in range(5):
            hs.append(pltpu.async_copy(
                face_h.at[j, img, pl.ds(base, CH)], fv.at[j], sem))
        for j in range(10):
            hs.append(pltpu.async_copy(
                lmp_h.at[j, img, pl.ds(base, CH)], lv.at[j], sem))
        hs.append(pltpu.async_copy(
            gzp_h.at[img, pl.ds(base * 2, CH * 2)], gv, sem))
        hs.append(pltpu.async_copy(mat_h.at[img, pl.ds(base, CH)], mv, sem))
        hs.append(pltpu.async_copy(lab_h.at[img, pl.ds(base, CH)], bv, sem))
        return hs

    iota = jnp.arange(L, dtype=jnp.int32)
    i2 = iota * 2

    def compute(slot, accs):
        fv, lv, gv, mv, bv, _ = bufs[slot]

        def group(g, accs):
            abce, abox, alm, agz = accs
            off = g * L
            m = mv[pl.ds(off, L)]
            lab = bv[pl.ds(off, L)]
            maskf = jnp.where(lab > 0.0, 1.0, 0.0).astype(jnp.float32)

            def sl1_plane(pred_ref, tbl_ref, comps):
                sq = jnp.zeros((L,), jnp.float32)
                mx = jnp.zeros((L,), jnp.float32)
                for j in comps:
                    p = pred_ref[j, pl.ds(off, L)]
                    t = plsc.load_gather(tbl_ref, [jnp.full((L,), j, jnp.int32), m])
                    d = jnp.abs(p - t)
                    dm = jnp.minimum(d, 1.0)
                    sq = sq + dm * dm
                    mx = mx + jnp.maximum(d, 1.0)
                return (0.5 * sq + mx - float(len(comps))) * maskf

            abox = abox + sl1_plane(fv, tbox_v, range(4))
            alm = alm + sl1_plane(lv, tlm_v, range(10))

            sq = jnp.zeros((L,), jnp.float32)
            mx = jnp.zeros((L,), jnp.float32)
            off2 = off * 2
            for j in range(2):
                p = plsc.load_gather(gv, [off2 + i2 + j])
                t = plsc.load_gather(tgz_v, [jnp.full((L,), j, jnp.int32), m])
                d = jnp.abs(p - t)
                dm = jnp.minimum(d, 1.0)
                sq = sq + dm * dm
                mx = mx + jnp.maximum(d, 1.0)
            agz = agz + (0.5 * sq + mx - 2.0) * maskf

            x = fv[4, pl.ds(off, L)]
            u = jnp.exp(-jnp.abs(x))
            v = u / (u + 2.0)
            v2 = v * v
            sp = v * (2.0 + v2 * (2.0 / 3.0 + v2 * (2.0 / 5.0
                      + v2 * (2.0 / 7.0 + v2 * (2.0 / 9.0)))))
            abce = abce + (jnp.maximum(x, 0.0) - x * lab + sp)
            return (abce, abox, alm, agz)

        return lax.fori_loop(0, GRP, group, accs, unroll=4)

    z = jnp.zeros((L,), jnp.float32)
    accs = (z, z, z, z)
    pending = start(0, 0)
    for c in range(NCHUNK):
        for hd in pending:
            hd.wait()
        if c + 1 < NCHUNK:
            nxt = start(c + 1, (c + 1) % 2)
        else:
            nxt = []
        accs = compute(c % 2, accs)
        pending = nxt

    out_v[0, :] = accs[0]
    out_v[1, :] = accs[1]
    out_v[2, :] = accs[2]
    out_v[3, :] = accs[3]
    pltpu.sync_copy(out_v, out_h.at[wid])


_sc_loss = functools.partial(
    pl.kernel,
    out_type=jax.ShapeDtypeStruct((NW, 4, L), jnp.float32),
    mesh=_mesh,
    scratch_types=[
        pltpu.VMEM((5, CH), jnp.float32),
        pltpu.VMEM((5, CH), jnp.float32),
        pltpu.VMEM((10, CH), jnp.float32),
        pltpu.VMEM((10, CH), jnp.float32),
        pltpu.VMEM((CH * 2,), jnp.float32),
        pltpu.VMEM((CH * 2,), jnp.float32),
        pltpu.VMEM((CH,), jnp.int32),
        pltpu.VMEM((CH,), jnp.int32),
        pltpu.VMEM((CH,), jnp.float32),
        pltpu.VMEM((CH,), jnp.float32),
        pltpu.VMEM((4, M), jnp.float32),
        pltpu.VMEM((10, M), jnp.float32),
        pltpu.VMEM((2, M), jnp.float32),
        pltpu.VMEM((4, L), jnp.float32),
        pltpu.SemaphoreType.DMA,
        pltpu.SemaphoreType.DMA,
    ],
    compiler_params=pltpu.CompilerParams(
        needs_layout_passes=False, use_tc_tiling_on_sc=False),
)(_body)


def kernel(face_preds, landmark_preds, gaze_preds, boxes, landmarks, gaze,
           matches, labels):
    part = _sc_loss(
        jnp.transpose(face_preds, (2, 0, 1)),
        jnp.transpose(landmark_preds, (2, 0, 1)),
        gaze_preds.reshape(B, N * 2),
        jnp.transpose(boxes, (0, 2, 1)),
        jnp.transpose(landmarks, (2, 0, 1)),
        jnp.transpose(gaze, (0, 2, 1)),
        matches.astype(jnp.int32),
        labels,
    )
    s = jnp.sum(part, axis=(0, 2))
    face_loss = s[0] + s[1]
    landmark_loss = s[2]
    gaze_loss = s[3]
    total_loss = face_loss + landmark_loss + gaze_loss
    return (total_loss, face_loss, landmark_loss, gaze_loss)


# speedup vs baseline: 8.4884x; 1.4503x over previous
import functools

import jax
import jax.numpy as jnp
from jax import lax
from jax.experimental import pallas as pl
from jax.experimental.pallas import tpu as pltpu
from jax.experimental.pallas import tpu_sc as plsc

B = 16
N = 16384
M = 64
L = 16
NC = 2
NS = 16
NW = NC * NS
APW = (B * N) // NW
CH = 2048
NCHUNK = APW // CH
GRP = CH // L

_mesh = plsc.VectorSubcoreMesh(core_axis_name="c", subcore_axis_name="s")


def _body(face_h, lmp_h, gzp_h, tbox_h, tlm_h, tgz_h, mat_h, lab_h, out_h,
          face_v0, face_v1, lmp_v0, lmp_v1, gzp_v0, gzp_v1,
          mat_v0, mat_v1, lab_v0, lab_v1,
          tbox_v, tlm_v, tgz_v, out_v, sem0, sem1):
    cid = lax.axis_index("c")
    sid = lax.axis_index("s")
    wid = sid * NC + cid
    img = wid // 2
    half = wid % 2
    n0 = half * APW

    pltpu.sync_copy(tbox_h.at[img], tbox_v)
    pltpu.sync_copy(tlm_h.at[img], tlm_v)
    pltpu.sync_copy(tgz_h.at[img], tgz_v)

    bufs = ((face_v0, lmp_v0, gzp_v0, mat_v0, lab_v0, sem0),
            (face_v1, lmp_v1, gzp_v1, mat_v1, lab_v1, sem1))

    def start(c, slot):
        fv, lv, gv, mv, bv, sem = bufs[slot]
        base = n0 + c * CH
        hs = []
        for j in range(5):
            hs.append(pltpu.async_copy(
                face_h.at[j, img, pl.ds(base, CH)], fv.at[pl.ds(j * CH, CH)], sem))
        for j in range(10):
            hs.append(pltpu.async_copy(
                lmp_h.at[j, img, pl.ds(base, CH)], lv.at[pl.ds(j * CH, CH)], sem))
        hs.append(pltpu.async_copy(
            gzp_h.at[img, pl.ds(base * 2, CH * 2)], gv, sem))
        hs.append(pltpu.async_copy(mat_h.at[img, pl.ds(base, CH)], mv, sem))
        hs.append(pltpu.async_copy(lab_h.at[img, pl.ds(base, CH)], bv, sem))
        return hs

    iota = jnp.arange(L, dtype=jnp.int32)
    if5 = [iota + j * CH for j in range(5)]
    if10 = [iota + j * CH for j in range(10)]
    i2j = [iota * 2 + j for j in range(2)]
    it4 = [iota * 0 + j * M for j in range(4)]
    it10 = [iota * 0 + j * M for j in range(10)]
    it2 = [iota * 0 + j * M for j in range(2)]

    def compute(slot, accs):
        fv, lv, gv, mv, bv, _ = bufs[slot]

        def group(g, accs):
            abce, abox, alm, agz = accs
            off = g * L
            aidx = off + iota
            m = plsc.load_gather(mv, [aidx])
            lab = plsc.load_gather(bv, [aidx])
            maskf = jnp.where(lab > 0.0, 1.0, 0.0).astype(jnp.float32)

            def sl1(pred_ref, p_bases, tbl_ref, t_bases, ncomp):
                sq = jnp.zeros((L,), jnp.float32)
                mx = jnp.zeros((L,), jnp.float32)
                for j in range(ncomp):
                    p = plsc.load_gather(pred_ref, [off + p_bases[j]])
                    t = plsc.load_gather(tbl_ref, [m + t_bases[j]])
                    d = jnp.abs(p - t)
                    dm = jnp.minimum(d, 1.0)
                    sq = sq + dm * dm
                    mx = mx + jnp.maximum(d, 1.0)
                return (0.5 * sq + mx - float(ncomp)) * maskf

            abox = abox + sl1(fv, if5, tbox_v, it4, 4)
            alm = alm + sl1(lv, if10, tlm_v, it10, 10)

            sq = jnp.zeros((L,), jnp.float32)
            mx = jnp.zeros((L,), jnp.float32)
            off2 = off * 2
            for j in range(2):
                p = plsc.load_gather(gv, [off2 + i2j[j]])
                t = plsc.load_gather(tgz_v, [m + it2[j]])
                d = jnp.abs(p - t)
                dm = jnp.minimum(d, 1.0)
                sq = sq + dm * dm
                mx = mx + jnp.maximum(d, 1.0)
            agz = agz + (0.5 * sq + mx - 2.0) * maskf

            x = plsc.load_gather(fv, [off + if5[4]])
            u = jnp.exp(-jnp.abs(x))
            v = u / (u + 2.0)
            v2 = v * v
            sp = v * (2.0 + v2 * (2.0 / 3.0 + v2 * (2.0 / 5.0
                      + v2 * (2.0 / 7.0 + v2 * (2.0 / 9.0)))))
            abce = abce + (jnp.maximum(x, 0.0) - x * lab + sp)
            return (abce, abox, alm, agz)

        return lax.fori_loop(0, GRP, group, accs)

    z = jnp.zeros((L,), jnp.float32)
    accs = (z, z, z, z)
    pending = start(0, 0)
    for c in range(NCHUNK):
        for hd in pending:
            hd.wait()
        if c + 1 < NCHUNK:
            nxt = start(c + 1, (c + 1) % 2)
        else:
            nxt = []
        accs = compute(c % 2, accs)
        pending = nxt

    out_v[0, :] = accs[0]
    out_v[1, :] = accs[1]
    out_v[2, :] = accs[2]
    out_v[3, :] = accs[3]
    pltpu.sync_copy(out_v, out_h.at[wid])


_sc_loss = functools.partial(
    pl.kernel,
    out_type=jax.ShapeDtypeStruct((NW, 4, L), jnp.float32),
    mesh=_mesh,
    scratch_types=[
        pltpu.VMEM((5 * CH,), jnp.float32),
        pltpu.VMEM((5 * CH,), jnp.float32),
        pltpu.VMEM((10 * CH,), jnp.float32),
        pltpu.VMEM((10 * CH,), jnp.float32),
        pltpu.VMEM((CH * 2,), jnp.float32),
        pltpu.VMEM((CH * 2,), jnp.float32),
        pltpu.VMEM((CH,), jnp.int32),
        pltpu.VMEM((CH,), jnp.int32),
        pltpu.VMEM((CH,), jnp.float32),
        pltpu.VMEM((CH,), jnp.float32),
        pltpu.VMEM((4 * M,), jnp.float32),
        pltpu.VMEM((10 * M,), jnp.float32),
        pltpu.VMEM((2 * M,), jnp.float32),
        pltpu.VMEM((4, L), jnp.float32),
        pltpu.SemaphoreType.DMA,
        pltpu.SemaphoreType.DMA,
    ],
    compiler_params=pltpu.CompilerParams(
        needs_layout_passes=False, use_tc_tiling_on_sc=False),
)(_body)


def kernel(face_preds, landmark_preds, gaze_preds, boxes, landmarks, gaze,
           matches, labels):
    part = _sc_loss(
        jnp.transpose(face_preds, (2, 0, 1)),
        jnp.transpose(landmark_preds, (2, 0, 1)),
        gaze_preds.reshape(B, N * 2),
        jnp.transpose(boxes, (0, 2, 1)).reshape(B, 4 * M),
        jnp.transpose(landmarks, (0, 2, 1)).reshape(B, 10 * M),
        jnp.transpose(gaze, (0, 2, 1)).reshape(B, 2 * M),
        matches.astype(jnp.int32),
        labels,
    )
    s = jnp.sum(part, axis=(0, 2))
    face_loss = s[0] + s[1]
    landmark_loss = s[2]
    gaze_loss = s[3]
    total_loss = face_loss + landmark_loss + gaze_loss
    return (total_loss, face_loss, landmark_loss, gaze_loss)


# speedup vs baseline: 11.4857x; 1.3531x over previous
import functools

import jax
import jax.numpy as jnp
from jax import lax
from jax.experimental import pallas as pl
from jax.experimental.pallas import tpu as pltpu
from jax.experimental.pallas import tpu_sc as plsc

B = 16
N = 16384
M = 64
L = 16
NC = 2
NS = 16
NW = NC * NS
APW = (B * N) // NW
CH = 2048
NCHUNK = APW // CH
GRP = CH // L

_mesh = plsc.VectorSubcoreMesh(core_axis_name="c", subcore_axis_name="s")


def _body(face_h, lmp_h, gzp_h, tbox_h, tlm_h, tgz_h, mat_h, lab_h, out_h,
          face_v0, face_v1, lmp_v0, lmp_v1, gzp_v0, gzp_v1,
          mat_v0, mat_v1, lab_v0, lab_v1,
          tbox_v, tlm_v, tgz_v, out_v, sem0, sem1):
    cid = lax.axis_index("c")
    sid = lax.axis_index("s")
    wid = sid * NC + cid
    img = wid // 2
    half = wid % 2
    n0 = half * APW

    pltpu.sync_copy(tbox_h.at[img], tbox_v)
    pltpu.sync_copy(tlm_h.at[img], tlm_v)
    pltpu.sync_copy(tgz_h.at[img], tgz_v)

    bufs = ((face_v0, lmp_v0, gzp_v0, mat_v0, lab_v0, sem0),
            (face_v1, lmp_v1, gzp_v1, mat_v1, lab_v1, sem1))

    def start(c, slot):
        fv, lv, gv, mv, bv, sem = bufs[slot]
        base = n0 + c * CH
        hs = []
        for j in range(5):
            hs.append(pltpu.async_copy(
                face_h.at[j, img, pl.ds(base, CH)], fv.at[pl.ds(j * CH, CH)], sem))
        for j in range(10):
            hs.append(pltpu.async_copy(
                lmp_h.at[j, img, pl.ds(base, CH)], lv.at[pl.ds(j * CH, CH)], sem))
        hs.append(pltpu.async_copy(
            gzp_h.at[img, pl.ds(base // 128, CH // 128), :, :], gv, sem))
        hs.append(pltpu.async_copy(mat_h.at[img, pl.ds(base, CH)], mv, sem))
        hs.append(pltpu.async_copy(lab_h.at[img, pl.ds(base, CH)], bv, sem))
        return hs

    iota = jnp.arange(L, dtype=jnp.int32)
    if5 = [iota + j * CH for j in range(5)]
    if10 = [iota + j * CH for j in range(10)]
    izero = iota * 0
    it4 = [iota * 0 + j * M for j in range(4)]
    it10 = [iota * 0 + j * M for j in range(10)]
    it2 = [iota * 0 + j * M for j in range(2)]

    def compute(slot, accs):
        fv, lv, gv, mv, bv, _ = bufs[slot]

        def group(g, accs):
            abce, abox, alm, agz = accs
            off = g * L
            aidx = off + iota
            m = plsc.load_gather(mv, [aidx])
            lab = plsc.load_gather(bv, [aidx])
            maskf = jnp.where(lab > 0.0, 1.0, 0.0).astype(jnp.float32)

            def sl1(pred_ref, p_bases, tbl_ref, t_bases, ncomp):
                sq = jnp.zeros((L,), jnp.float32)
                mx = jnp.zeros((L,), jnp.float32)
                for j in range(ncomp):
                    p = plsc.load_gather(pred_ref, [off + p_bases[j]])
                    t = plsc.load_gather(tbl_ref, [m + t_bases[j]])
                    d = jnp.abs(p - t)
                    dm = jnp.minimum(d, 1.0)
                    sq = sq + dm * dm
                    mx = mx + jnp.maximum(d, 1.0)
                return (0.5 * sq + mx - float(ncomp)) * maskf

            abox = abox + sl1(fv, if5, tbox_v, it4, 4)
            alm = alm + sl1(lv, if10, tlm_v, it10, 10)

            tcv = (g // 8) + izero
            colv = (g % 8) * L + iota
            sq = jnp.zeros((L,), jnp.float32)
            mx = jnp.zeros((L,), jnp.float32)
            for j in range(2):
                p = plsc.load_gather(gv, [tcv, izero + j, colv])
                t = plsc.load_gather(tgz_v, [m + it2[j]])
                d = jnp.abs(p - t)
                dm = jnp.minimum(d, 1.0)
                sq = sq + dm * dm
                mx = mx + jnp.maximum(d, 1.0)
            agz = agz + (0.5 * sq + mx - 2.0) * maskf

            x = plsc.load_gather(fv, [off + if5[4]])
            u = jnp.exp(-jnp.abs(x))
            v = u / (u + 2.0)
            v2 = v * v
            sp = v * (2.0 + v2 * (2.0 / 3.0 + v2 * (2.0 / 5.0
                      + v2 * (2.0 / 7.0 + v2 * (2.0 / 9.0)))))
            abce = abce + (jnp.maximum(x, 0.0) - x * lab + sp)
            return (abce, abox, alm, agz)

        return lax.fori_loop(0, GRP, group, accs)

    z = jnp.zeros((L,), jnp.float32)
    accs = (z, z, z, z)
    pending = start(0, 0)
    for c in range(NCHUNK):
        for hd in pending:
            hd.wait()
        if c + 1 < NCHUNK:
            nxt = start(c + 1, (c + 1) % 2)
        else:
            nxt = []
        accs = compute(c % 2, accs)
        pending = nxt

    out_v[0, :] = accs[0]
    out_v[1, :] = accs[1]
    out_v[2, :] = accs[2]
    out_v[3, :] = accs[3]
    pltpu.sync_copy(out_v, out_h.at[wid])


_sc_loss = functools.partial(
    pl.kernel,
    out_type=jax.ShapeDtypeStruct((NW, 4, L), jnp.float32),
    mesh=_mesh,
    scratch_types=[
        pltpu.VMEM((5 * CH,), jnp.float32),
        pltpu.VMEM((5 * CH,), jnp.float32),
        pltpu.VMEM((10 * CH,), jnp.float32),
        pltpu.VMEM((10 * CH,), jnp.float32),
        pltpu.VMEM((CH // 128, 2, 128), jnp.float32),
        pltpu.VMEM((CH // 128, 2, 128), jnp.float32),
        pltpu.VMEM((CH,), jnp.int32),
        pltpu.VMEM((CH,), jnp.int32),
        pltpu.VMEM((CH,), jnp.float32),
        pltpu.VMEM((CH,), jnp.float32),
        pltpu.VMEM((4 * M,), jnp.float32),
        pltpu.VMEM((10 * M,), jnp.float32),
        pltpu.VMEM((2 * M,), jnp.float32),
        pltpu.VMEM((4, L), jnp.float32),
        pltpu.SemaphoreType.DMA,
        pltpu.SemaphoreType.DMA,
    ],
    compiler_params=pltpu.CompilerParams(
        needs_layout_passes=False, use_tc_tiling_on_sc=False),
)(_body)


def kernel(face_preds, landmark_preds, gaze_preds, boxes, landmarks, gaze,
           matches, labels):
    part = _sc_loss(
        jnp.transpose(face_preds, (2, 0, 1)),
        jnp.transpose(landmark_preds, (2, 0, 1)),
        gaze_preds.reshape(B, 128, 128, 2).transpose(0, 1, 3, 2),
        jnp.transpose(boxes, (0, 2, 1)).reshape(B, 4 * M),
        jnp.transpose(landmarks, (0, 2, 1)).reshape(B, 10 * M),
        jnp.transpose(gaze, (0, 2, 1)).reshape(B, 2 * M),
        matches.astype(jnp.int32),
        labels,
    )
    s = jnp.sum(part, axis=(0, 2))
    face_loss = s[0] + s[1]
    landmark_loss = s[2]
    gaze_loss = s[3]
    total_loss = face_loss + landmark_loss + gaze_loss
    return (total_loss, face_loss, landmark_loss, gaze_loss)


# speedup vs baseline: 13.0919x; 1.1398x over previous
import functools

import jax
import jax.numpy as jnp
from jax import lax
from jax.experimental import pallas as pl
from jax.experimental.pallas import tpu as pltpu
from jax.experimental.pallas import tpu_sc as plsc

B = 16
N = 16384
M = 64
L = 16
NC = 2
NS = 16
NW = NC * NS
TRW = B * N // 2
SPW = TRW // 16
CH = 2048
NCHUNK = SPW // CH
GRP = CH // L
GT = CH // 1024

_mesh = plsc.VectorSubcoreMesh(core_axis_name="c", subcore_axis_name="s")


def _body(face_h, lmp_h, gzp_h, tbox_h, tlm_h, tgz_h, mat_h, lab_h, out_h,
          face_v0, face_v1, lmp_v0, lmp_v1, gzp_v0, gzp_v1,
          mat_v0, mat_v1, lab_v0, lab_v1,
          tbox_v, tlm_v, tgz_v, out_v, sem0, sem1):
    cid = lax.axis_index("c")
    sid = lax.axis_index("s")
    wid = sid * NC + cid
    tr = wid // 16
    stripe = wid % 16
    w0 = stripe * SPW

    pltpu.sync_copy(tbox_h.at[pl.ds(tr * 8, 8)], tbox_v)
    pltpu.sync_copy(tlm_h.at[pl.ds(tr * 8, 8)], tlm_v)
    pltpu.sync_copy(tgz_h.at[pl.ds(tr * 8, 8)], tgz_v)

    bufs = ((face_v0, lmp_v0, gzp_v0, mat_v0, lab_v0, sem0),
            (face_v1, lmp_v1, gzp_v1, mat_v1, lab_v1, sem1))

    def start(c, slot):
        fv, lv, gv, mv, bv, sem = bufs[slot]
        base = w0 + c * CH
        gz0 = (base // 1024) * 256
        hs = []
        for j in range(5):
            hs.append(pltpu.async_copy(
                face_h.at[j, tr, pl.ds(base, CH)], fv.at[pl.ds(j * CH, CH)], sem))
        for j in range(10):
            hs.append(pltpu.async_copy(
                lmp_h.at[j, tr, pl.ds(base, CH)], lv.at[pl.ds(j * CH, CH)], sem))
        for r in range(8):
            hs.append(pltpu.async_copy(
                gzp_h.at[tr * 8 + r, pl.ds(gz0, GT * 256)], gv.at[r], sem))
        hs.append(pltpu.async_copy(mat_h.at[tr, pl.ds(base, CH)], mv, sem))
        hs.append(pltpu.async_copy(lab_h.at[tr, pl.ds(base, CH)], bv, sem))
        return hs

    iota = jnp.arange(L, dtype=jnp.int32)
    izero = iota * 0
    if5 = [iota + j * CH for j in range(5)]
    if10 = [iota + j * CH for j in range(10)]
    it4 = [izero + j * M for j in range(4)]
    it10 = [izero + j * M for j in range(10)]
    it2 = [izero + j * M for j in range(2)]
    ig2 = [iota + j * 128 for j in range(2)]

    def compute(slot, accs):
        fv, lv, gv, mv, bv, _ = bufs[slot]

        def group(g, accs):
            abce, abox, alm, agz = accs
            off = g * L
            row = (g // 8) % 8
            rowv = row + izero
            aidx = off + iota
            m = plsc.load_gather(mv, [aidx])
            lab = plsc.load_gather(bv, [aidx])
            maskf = jnp.where(lab > 0.0, 1.0, 0.0).astype(jnp.float32)

            def sl1(pred_ref, p_bases, tbl_ref, t_bases, ncomp):
                sq = jnp.zeros((L,), jnp.float32)
                mx = jnp.zeros((L,), jnp.float32)
                for j in range(ncomp):
                    p = plsc.load_gather(pred_ref, [off + p_bases[j]])
                    t = plsc.load_gather(tbl_ref, [rowv, m + t_bases[j]])
                    d = jnp.abs(p - t)
                    dm = jnp.minimum(d, 1.0)
                    sq = sq + dm * dm
                    mx = mx + jnp.maximum(d, 1.0)
                return (0.5 * sq + mx - float(ncomp)) * maskf

            abox = abox + sl1(fv, if5, tbox_v, it4, 4)
            alm = alm + sl1(lv, if10, tlm_v, it10, 10)

            gb = (g // 64) * 256 + (g % 8) * L
            sq = jnp.zeros((L,), jnp.float32)
            mx = jnp.zeros((L,), jnp.float32)
            for j in range(2):
                p = plsc.load_gather(gv, [rowv, gb + ig2[j]])
                t = plsc.load_gather(tgz_v, [rowv, m + it2[j]])
                d = jnp.abs(p - t)
                dm = jnp.minimum(d, 1.0)
                sq = sq + dm * dm
                mx = mx + jnp.maximum(d, 1.0)
            agz = agz + (0.5 * sq + mx - 2.0) * maskf

            x = plsc.load_gather(fv, [off + if5[4]])
            u = jnp.exp(-jnp.abs(x))
            v = u / (u + 2.0)
            v2 = v * v
            sp = v * (2.0 + v2 * (2.0 / 3.0 + v2 * (2.0 / 5.0
                      + v2 * (2.0 / 7.0 + v2 * (2.0 / 9.0)))))
            abce = abce + (jnp.maximum(x, 0.0) - x * lab + sp)
            return (abce, abox, alm, agz)

        return lax.fori_loop(0, GRP, group, accs)

    z = jnp.zeros((L,), jnp.float32)
    accs = (z, z, z, z)
    pending = start(0, 0)
    for c in range(NCHUNK):
        for hd in pending:
            hd.wait()
        if c + 1 < NCHUNK:
            nxt = start(c + 1, (c + 1) % 2)
        else:
            nxt = []
        accs = compute(c % 2, accs)
        pending = nxt

    out_v[0, :] = accs[0]
    out_v[1, :] = accs[1]
    out_v[2, :] = accs[2]
    out_v[3, :] = accs[3]
    pltpu.sync_copy(out_v, out_h.at[wid])


_sc_loss = functools.partial(
    pl.kernel,
    out_type=jax.ShapeDtypeStruct((NW, 4, L), jnp.float32),
    mesh=_mesh,
    scratch_types=[
        pltpu.VMEM((5 * CH,), jnp.float32),
        pltpu.VMEM((5 * CH,), jnp.float32),
        pltpu.VMEM((10 * CH,), jnp.float32),
        pltpu.VMEM((10 * CH,), jnp.float32),
        pltpu.VMEM((8, GT * 256), jnp.float32),
        pltpu.VMEM((8, GT * 256), jnp.float32),
        pltpu.VMEM((CH,), jnp.int32),
        pltpu.VMEM((CH,), jnp.int32),
        pltpu.VMEM((CH,), jnp.float32),
        pltpu.VMEM((CH,), jnp.float32),
        pltpu.VMEM((8, 4 * M), jnp.float32),
        pltpu.VMEM((8, 10 * M), jnp.float32),
        pltpu.VMEM((8, 2 * M), jnp.float32),
        pltpu.VMEM((4, L), jnp.float32),
        pltpu.SemaphoreType.DMA,
        pltpu.SemaphoreType.DMA,
    ],
    compiler_params=pltpu.CompilerParams(
        needs_layout_passes=False, use_tc_tiling_on_sc=False),
)(_body)


def _tile_view(x):
    c = x.shape[2]
    return (x.transpose(2, 0, 1)
             .reshape(c, 2, 8, 128, 128)
             .transpose(0, 1, 3, 2, 4)
             .reshape(c, 2, TRW))


def _tile_view2(x):
    return (x.reshape(2, 8, 128, 128)
             .transpose(0, 2, 1, 3)
             .reshape(2, TRW))


def kernel(face_preds, landmark_preds, gaze_preds, boxes, landmarks, gaze,
           matches, labels):
    part = _sc_loss(
        _tile_view(face_preds),
        _tile_view(landmark_preds),
        gaze_preds.reshape(B, 128, 128, 2).transpose(0, 1, 3, 2).reshape(B, 32768),
        jnp.transpose(boxes, (0, 2, 1)).reshape(B, 4 * M),
        jnp.transpose(landmarks, (0, 2, 1)).reshape(B, 10 * M),
        jnp.transpose(gaze, (0, 2, 1)).reshape(B, 2 * M),
        _tile_view2(matches.astype(jnp.int32)),
        _tile_view2(labels),
    )
    s = jnp.sum(part, axis=(0, 2))
    face_loss = s[0] + s[1]
    landmark_loss = s[2]
    gaze_loss = s[3]
    total_loss = face_loss + landmark_loss + gaze_loss
    return (total_loss, face_loss, landmark_loss, gaze_loss)
